# baseline (speedup 1.0000x reference)
import functools

import jax
import jax.numpy as jnp
from jax import lax
from jax.experimental import pallas as pl
from jax.experimental.pallas import tpu as pltpu
from jax.experimental.pallas import tpu_sc as plsc

_N = 10000
_E = 160000
_NR = 4
_NBK = 4
_BI = 75
_BO = 128
_EP = 163840
_NC = 2
_NS = 16
_ET = _EP // _NS
_BQ = 64
_STRIP = 2048
_CAP = _ET + 128
_BKT = _CAP + 8
_CW = 128
_NCK = 3
_RPT = 624
_AGG_ROWS = _N + 8
_ZR = 48


def _sc_body(x0, x1, x2, srcf, dstf, etf, zsrc, zdst, zf, outS,
             src_bkt, dst_bkt, s_strip, d_strip, e_strip, didx, zbuf,
             rows_a, rows_b, agg_sh, sem_a, sem_b):
    c2 = lax.axis_index("c")
    s = lax.axis_index("s")
    e0 = s * _ET
    r0 = s * _RPT
    tail = _N - _NS * _RPT
    ra = c2 * 2

    pltpu.sync_copy(zsrc, src_bkt)
    pltpu.sync_copy(zdst, dst_bkt)
    pltpu.sync_copy(zf, zbuf)

    def strip_body(k, bases):
        off = e0 + k * _STRIP
        pltpu.sync_copy(srcf.at[pl.ds(off, _STRIP)], s_strip)
        pltpu.sync_copy(dstf.at[pl.ds(off, _STRIP)], d_strip)
        pltpu.sync_copy(etf.at[pl.ds(off, _STRIP)], e_strip)

        def vec_body(i, b):
            bA, bB = b
            o = i * 16
            ev = e_strip[pl.ds(o, 16)]
            sv = s_strip[pl.ds(o, 16)]
            dv = d_strip[pl.ds(o, 16)]
            mA = ev == ra
            onesA = jnp.where(mA, 1, 0).astype(jnp.int32)
            cA = plsc.cumsum(onesA)
            posA = jnp.where(mA, bA + cA - 1, _CAP)
            plsc.store_scatter(src_bkt, [posA], sv, mask=mA)
            plsc.store_scatter(dst_bkt, [posA], dv, mask=mA)
            mB = ev == ra + 1
            onesB = jnp.where(mB, 1, 0).astype(jnp.int32)
            cB = plsc.cumsum(onesB)
            posB = jnp.where(mB, (_CAP - 1) - (bB + cB - 1), _CAP + 1)
            plsc.store_scatter(src_bkt, [posB], sv, mask=mB)
            plsc.store_scatter(dst_bkt, [posB], dv, mask=mB)
            return (bA + jnp.sum(onesA), bB + jnp.sum(onesB))

        return lax.fori_loop(0, _STRIP // 16, vec_body, bases)

    cntA, cntB = lax.fori_loop(0, _ET // _STRIP, strip_body,
                               (jnp.int32(0), jnp.int32(0)))

    for rl in range(2):
        r = ra + rl
        cnt = cntA if rl == 0 else cntB
        nb = (cnt + _BQ - 1) // _BQ
        npairs = (nb + 1) // 2
        for ck in range(_NCK):
            xt = (x0, x1, x2)[ck]

            for z in range(_RPT // _ZR):
                pltpu.sync_copy(zbuf, agg_sh.at[pl.ds(r0 + z * _ZR, _ZR)])

            @pl.when(s == _NS - 1)
            def _():
                pltpu.sync_copy(zbuf.at[pl.ds(0, tail)],
                                agg_sh.at[pl.ds(_NS * _RPT, tail)])

            plsc.subcore_barrier()

            def boff(b):
                if rl == 0:
                    return b * _BQ
                return _CAP - _BQ - b * _BQ

            def fire(b, buf, sem):
                pltpu.async_copy(xt.at[src_bkt.at[pl.ds(boff(b), _BQ)]],
                                 buf, sem)

            def drain(buf, sem):
                pltpu.make_async_copy(xt.at[src_bkt.at[pl.ds(0, _BQ)]],
                                      buf, sem).wait()

            def scat(b, buf, k):
                def cp(j, carry):
                    didx[k, pl.ds(j * 16, 16)] = \
                        dst_bkt[pl.ds(boff(b) + j * 16, 16)]
                    return carry
                lax.fori_loop(0, _BQ // 16, cp, 0)
                pltpu.sync_copy(buf, agg_sh.at[didx.at[k]], add=True)

            @pl.when(nb > 0)
            def _():
                fire(0, rows_a, sem_a)

            def pbody(j, carry):
                b0 = j * 2
                drain(rows_a, sem_a)

                @pl.when(b0 + 1 < nb)
                def _():
                    fire(b0 + 1, rows_b, sem_b)

                scat(b0, rows_a, 0)

                @pl.when(b0 + 2 < nb)
                def _():
                    fire(b0 + 2, rows_a, sem_a)

                @pl.when(b0 + 1 < nb)
                def _():
                    drain(rows_b, sem_b)
                    scat(b0 + 1, rows_b, 1)

                return carry
            lax.fori_loop(0, npairs, pbody, 0)

            plsc.subcore_barrier()

            pltpu.sync_copy(agg_sh.at[pl.ds(r0, _RPT)],
                            outS.at[r, ck, pl.ds(r0, _RPT)])

            @pl.when(s == _NS - 1)
            def _():
                pltpu.sync_copy(agg_sh.at[pl.ds(_NS * _RPT, tail)],
                                outS.at[r, ck, pl.ds(_NS * _RPT, tail)])

            plsc.subcore_barrier()


def _sc_aggregate(x0, x1, x2, srcp, dstp, etp, zsrc, zdst, zf):
    mesh = plsc.VectorSubcoreMesh(core_axis_name="c", subcore_axis_name="s")
    run = functools.partial(
        pl.kernel,
        out_type=jax.ShapeDtypeStruct((_NR, _NCK, _N, _CW), jnp.float32),
        mesh=mesh,
        scratch_types=[
            pltpu.VMEM((_BKT,), jnp.int32),
            pltpu.VMEM((_BKT,), jnp.int32),
            pltpu.VMEM((_STRIP,), jnp.int32),
            pltpu.VMEM((_STRIP,), jnp.int32),
            pltpu.VMEM((_STRIP,), jnp.int32),
            pltpu.VMEM((2, _BQ), jnp.int32),
            pltpu.VMEM((_ZR, _CW), jnp.float32),
            pltpu.VMEM((_BQ, _CW), jnp.float32),
            pltpu.VMEM((_BQ, _CW), jnp.float32),
            pltpu.VMEM_SHARED((_AGG_ROWS, _CW), jnp.float32),
            pltpu.SemaphoreType.DMA,
            pltpu.SemaphoreType.DMA,
        ],
        compiler_params=pltpu.CompilerParams(needs_layout_passes=False),
    )(_sc_body)
    return run(x0, x1, x2, srcp, dstp, etp, zsrc, zdst, zf)


_PIECES = {
    0: [(0, 0, 75, 0)],
    1: [(0, 75, 128, 0), (1, 0, 22, 53)],
    2: [(1, 22, 97, 0)],
    3: [(2, 53, 128, 0)],
}


def _tc_post_body(x_ref, s_ref, wb_ref, wl_ref, b_ref, o_ref):
    for co in range(_NBK):
        acc = jnp.dot(x_ref[...], wl_ref[:, co * 128:(co + 1) * 128],
                      preferred_element_type=jnp.float32,
                      precision=lax.Precision.HIGHEST)
        acc = acc + b_ref[0:1, co * 128:(co + 1) * 128]
        for r in range(_NR):
            for (p, a, b, wa) in _PIECES[co]:
                acc = acc + jnp.dot(
                    s_ref[r, p][:, a:b],
                    wb_ref[r, co][wa:wa + (b - a), :],
                    preferred_element_type=jnp.float32,
                    precision=lax.Precision.HIGHEST)
        o_ref[:, co * 128:(co + 1) * 128] = acc


def _tc_post(init_emb, S, w_bdd, w_loop, bias8):
    tn = 400
    grid = (_N // tn,)
    return pl.pallas_call(
        _tc_post_body,
        grid=grid,
        in_specs=[
            pl.BlockSpec((tn, 300), lambda t: (t, 0)),
            pl.BlockSpec((_NR, _NCK, tn, _CW), lambda t: (0, 0, t, 0)),
            pl.BlockSpec((_NR, _NBK, _BI, _BO), lambda t: (0, 0, 0, 0)),
            pl.BlockSpec((300, 512), lambda t: (0, 0)),
            pl.BlockSpec((8, 512), lambda t: (0, 0)),
        ],
        out_specs=pl.BlockSpec((tn, 512), lambda t: (t, 0)),
        out_shape=jax.ShapeDtypeStruct((_N, 512), jnp.float32),
    )(init_emb, S, w_bdd, w_loop, bias8)


def kernel(init_emb, W_bdd, W_loop, bias, edge_index, etype):
    bias8 = jnp.broadcast_to(bias.reshape(1, 512), (8, 512))
    x0 = init_emb[:, 0:128]
    x1 = init_emb[:, 128:256]
    x2 = init_emb[:, 172:300]

    src = edge_index[0]
    dst = edge_index[1]
    pad = _EP - _E
    srcp = jnp.concatenate([src, jnp.zeros((pad,), jnp.int32)])
    etp = jnp.concatenate([etype, jnp.zeros((pad,), jnp.int32)])
    dstp = jnp.concatenate([dst, jnp.full((pad,), _N, jnp.int32)])
    zsrc = jnp.zeros((_BKT,), jnp.int32)
    zdst = jnp.full((_BKT,), _N, jnp.int32)
    zf = jnp.zeros((_ZR, _CW), jnp.float32)

    S = _sc_aggregate(x0, x1, x2, srcp, dstp, etp, zsrc, zdst, zf)
    return _tc_post(init_emb, S, W_bdd, W_loop, bias8)
